# baseline (speedup 1.0000x reference)
import jax
import jax.numpy as jnp
from jax.experimental import pallas as pl


def _copy_body(x_ref, p_ref, xo_ref, po_ref):
    xo_ref[...] = x_ref[...]
    po_ref[...] = p_ref[...]


def kernel(x, positions, rel_pos_embeddings):
    B, S, D = x.shape
    ROWS = 1024
    grid_n = (B * S) // ROWS
    xr = x.reshape(B * S, D)
    npos = positions.size
    pr = positions.reshape(grid_n, 1, npos // grid_n)
    x_out, p_out = pl.pallas_call(
        _copy_body,
        grid=(grid_n,),
        in_specs=[
            pl.BlockSpec((ROWS, D), lambda i: (i, 0)),
            pl.BlockSpec((1, 1, npos // grid_n), lambda i: (i, 0, 0)),
        ],
        out_specs=[
            pl.BlockSpec((ROWS, D), lambda i: (i, 0)),
            pl.BlockSpec((1, 1, npos // grid_n), lambda i: (i, 0, 0)),
        ],
        out_shape=[
            jax.ShapeDtypeStruct((B * S, D), x.dtype),
            jax.ShapeDtypeStruct(pr.shape, positions.dtype),
        ],
    )(xr, pr)
    return (x_out.reshape(B, S, D), p_out.reshape(positions.shape))
